# Initial kernel scaffold; baseline (speedup 1.0000x reference)
import jax
import jax.numpy as jnp
from jax.experimental import pallas as pl


def kernel(flattened_logits, mask):
    raise NotImplementedError("write your pallas kernel here")


# speedup vs baseline: 8.0379x; 8.0379x over previous
import functools
import numpy as np
import jax
import jax.numpy as jnp
from jax import lax
from jax.experimental import pallas as pl
from jax.experimental.pallas import tpu as pltpu
from jax.experimental.pallas import tpu_sc as plsc

_B, _L = 16384, 200
_LOGEPS = np.float32(np.log(np.float32(1e-10)))
_NW = 32
_ROWS_PER_W = _B // _NW
_RCHUNK = 8
_NCHUNK = _ROWS_PER_W // _RCHUNK
_CL = _RCHUNK * _L
_NVEC = (_L + 15) // 16

_mesh = plsc.VectorSubcoreMesh(core_axis_name="c", subcore_axis_name="s")


@functools.partial(
    pl.kernel,
    out_type=jax.ShapeDtypeStruct((_B * _L,), jnp.float32),
    mesh=_mesh,
    scratch_types=[
        pltpu.VMEM((_CL + 416,), jnp.float32),
        pltpu.VMEM((_CL + 16,), jnp.int32),
        pltpu.VMEM((_CL + 16,), jnp.float32),
        pltpu.VMEM((208,), jnp.int32),
        pltpu.VMEM((208,), jnp.float32),
        pltpu.VMEM((208,), jnp.float32),
    ],
    compiler_params=pltpu.CompilerParams(needs_layout_passes=False),
)
def _restore(logits_hbm, mask_hbm, out_hbm, lbuf, mbuf, obuf, rankb, bsum, bmean):
    wid = lax.axis_index("s") * 2 + lax.axis_index("c")
    iota = lax.iota(jnp.int32, 16)
    zero16 = jnp.zeros((16,), jnp.float32)

    def row_body(r, _):
        ro = r * _L
        carry = jnp.int32(0)
        for jv in range(_NVEC):
            m = mbuf[pl.ds(ro + jv * 16, 16)]
            if jv == _NVEC - 1:
                m = jnp.where(iota < _L - (_NVEC - 1) * 16, m, 0)
            cum = plsc.cumsum(m)
            rankb[pl.ds(jv * 16, 16)] = cum - m + carry
            carry = carry + jnp.sum(m)
        nv = carry
        nv_safe = jnp.maximum(nv, 1)

        for kv in range(_NVEC):
            bsum[pl.ds(kv * 16, 16)] = zero16

        @pl.when(nv >= 16)
        def _():
            for jv in range(_NVEC):
                v = lbuf[pl.ds(ro + jv * 16, 16)]
                col = iota + jv * 16
                if jv == _NVEC - 1:
                    v = jnp.where(col < _L, v, 0.0)
                plsc.addupdate_scatter(bsum, [col % nv_safe], v)

        @pl.when(nv < 16)
        def _():
            qmax = jnp.int32(_L - 1) // nv_safe + 1

            def qstep(q, acc):
                off = q * nv_safe
                v = lbuf[pl.ds(ro + off, 16)]
                return acc + jnp.where(iota + off < _L, v, 0.0)

            bsum[pl.ds(0, 16)] = lax.fori_loop(0, qmax, qstep, zero16)

        for kv in range(_NVEC):
            k = iota + kv * 16
            cnt = jnp.maximum(_L - 1 - k, 0) // nv_safe + 1
            bmean[pl.ds(kv * 16, 16)] = bsum[pl.ds(kv * 16, 16)] / cnt.astype(jnp.float32)

        for jv in range(_NVEC):
            off = ro + jv * 16
            m = mbuf[pl.ds(off, 16)]
            g = plsc.load_gather(bmean, [rankb[pl.ds(jv * 16, 16)]])
            obuf[pl.ds(off, 16)] = jnp.where(m > 0, g, _LOGEPS)

        @pl.when(nv == 0)
        def _():
            v = obuf[pl.ds(ro, 16)]
            mv = bmean[pl.ds(0, 16)]
            obuf[pl.ds(ro, 16)] = jnp.where(iota == 0, mv, v)

        return 0

    def chunk_body(c, _):
        base = (wid * _ROWS_PER_W + c * _RCHUNK) * _L
        pltpu.sync_copy(logits_hbm.at[pl.ds(base, _CL)], lbuf.at[pl.ds(0, _CL)])
        pltpu.sync_copy(mask_hbm.at[pl.ds(base, _CL)], mbuf.at[pl.ds(0, _CL)])
        lax.fori_loop(0, _RCHUNK, row_body, 0)
        pltpu.sync_copy(obuf.at[pl.ds(0, _CL)], out_hbm.at[pl.ds(base, _CL)])
        return 0

    lax.fori_loop(0, _NCHUNK, chunk_body, 0)


def kernel(flattened_logits, mask):
    mask_i32 = mask.astype(jnp.int32).reshape(_B * _L)
    out = _restore(flattened_logits, mask_i32)
    return out.reshape(_B, _L)


# speedup vs baseline: 41.5146x; 5.1648x over previous
import functools
import numpy as np
import jax
import jax.numpy as jnp
from jax import lax
from jax.experimental import pallas as pl
from jax.experimental.pallas import tpu as pltpu
from jax.experimental.pallas import tpu_sc as plsc

_B, _L = 16384, 200
_LOGEPS = np.float32(np.log(np.float32(1e-10)))
_NW = 32
_ROWS_PER_W = _B // _NW
_RCHUNK = 32
_NCHUNK = _ROWS_PER_W // _RCHUNK
_CL = _RCHUNK * _L
_NVEC = (_L + 15) // 16

_mesh = plsc.VectorSubcoreMesh(core_axis_name="c", subcore_axis_name="s")


@functools.partial(
    pl.kernel,
    out_type=jax.ShapeDtypeStruct((_B * _L,), jnp.float32),
    mesh=_mesh,
    scratch_types=[
        pltpu.VMEM((_CL + 16,), jnp.float32),
        pltpu.VMEM((_CL + 16,), jnp.float32),
        pltpu.VMEM((_CL + 16,), jnp.int32),
        pltpu.VMEM((_CL + 16,), jnp.int32),
        pltpu.VMEM((_CL + 16,), jnp.float32),
        pltpu.VMEM((_CL + 16,), jnp.float32),
        pltpu.VMEM((208,), jnp.int32),
        pltpu.VMEM((208,), jnp.float32),
        pltpu.SemaphoreType.DMA,
        pltpu.SemaphoreType.DMA,
        pltpu.SemaphoreType.DMA,
        pltpu.SemaphoreType.DMA,
        pltpu.SemaphoreType.DMA,
        pltpu.SemaphoreType.DMA,
    ],
    compiler_params=pltpu.CompilerParams(needs_layout_passes=False),
)
def _restore(logits_hbm, mask_hbm, out_hbm,
             lbuf0, lbuf1, mbuf0, mbuf1, obuf0, obuf1, rankb, bsum,
             sl0, sl1, sm0, sm1, so0, so1):
    wid = lax.axis_index("s") * 2 + lax.axis_index("c")
    iota = lax.iota(jnp.int32, 16)
    zero16 = jnp.zeros((16,), jnp.float32)
    lbuf = (lbuf0, lbuf1)
    mbuf = (mbuf0, mbuf1)
    obuf = (obuf0, obuf1)
    sl = (sl0, sl1)
    sm = (sm0, sm1)
    so = (so0, so1)

    def chunk_base(c):
        return (wid * _ROWS_PER_W + c * _RCHUNK) * _L

    def start_in(c):
        p = c & 1
        base = chunk_base(c)
        hl = pltpu.async_copy(
            logits_hbm.at[pl.ds(base, _CL)], lbuf[p].at[pl.ds(0, _CL)], sl[p])
        hm = pltpu.async_copy(
            mask_hbm.at[pl.ds(base, _CL)], mbuf[p].at[pl.ds(0, _CL)], sm[p])
        return hl, hm

    def make_row_body(p):
        lb, mb, ob = lbuf[p], mbuf[p], obuf[p]

        def row_body(r, _):
            ro = r * _L
            carry = jnp.zeros((16,), jnp.int32)
            for jv in range(_NVEC):
                m = mb[pl.ds(ro + jv * 16, 16)]
                if jv == _NVEC - 1:
                    m = jnp.where(iota < _L - (_NVEC - 1) * 16, m, 0)
                cum = plsc.cumsum(m)
                rankb[pl.ds(jv * 16, 16)] = cum - m + carry
                carry = carry + plsc.all_reduce_population_count(m > 0)
            nv_vec = carry
            nv_safe = jnp.maximum(nv_vec, 1)
            is_fast = jnp.any(nv_vec >= 16)
            has_valid = jnp.any(nv_vec > 0)

            for kv in range(_NVEC):
                bsum[pl.ds(kv * 16, 16)] = zero16

            @pl.when(is_fast)
            def _():
                idx = iota
                for jv in range(_NVEC):
                    v = lb[pl.ds(ro + jv * 16, 16)]
                    if jv == _NVEC - 1:
                        v = jnp.where(iota + jv * 16 < _L, v, 0.0)
                    plsc.addupdate_scatter(bsum, [idx], v)
                    if jv != _NVEC - 1:
                        nxt = idx + 16
                        idx = jnp.where(nxt >= nv_vec, nxt - nv_vec, nxt)

            @pl.when(jnp.logical_not(is_fast))
            def _():
                nv_s = jnp.maximum(jnp.max(nv_vec), 1)
                qmax = jnp.int32(_L - 1) // nv_s + 1

                def qstep(q, acc):
                    off = q * nv_s
                    v = lb[pl.ds(ro + off, 16)]
                    return acc + jnp.where(iota + off < _L, v, 0.0)

                bsum[pl.ds(0, 16)] = lax.fori_loop(0, qmax, qstep, zero16)

            q_vec = jnp.full((16,), _L, jnp.int32) // nv_safe
            r_vec = _L - q_vec * nv_safe
            inv_hi = 1.0 / (q_vec + 1).astype(jnp.float32)
            inv_lo = 1.0 / q_vec.astype(jnp.float32)

            for jv in range(_NVEC):
                off = ro + jv * 16
                m = mb[pl.ds(off, 16)]
                rank = rankb[pl.ds(jv * 16, 16)]
                g = plsc.load_gather(bsum, [rank])
                inv = jnp.where(rank < r_vec, inv_hi, inv_lo)
                ob[pl.ds(off, 16)] = jnp.where(m > 0, g * inv, _LOGEPS)

            @pl.when(jnp.logical_not(has_valid))
            def _():
                v = ob[pl.ds(ro, 16)]
                mv = bsum[pl.ds(0, 16)] * inv_lo
                ob[pl.ds(ro, 16)] = jnp.where(iota == 0, mv, v)

            return 0

        return row_body

    in_handles = start_in(0)
    out_handles = [None, None]
    for c in range(_NCHUNK):
        p = c & 1
        for h in in_handles:
            h.wait()
        if c + 1 < _NCHUNK:
            in_handles = start_in(c + 1)
        if out_handles[p] is not None:
            out_handles[p].wait()
        lax.fori_loop(0, _RCHUNK, make_row_body(p), 0)
        out_handles[p] = pltpu.async_copy(
            obuf[p].at[pl.ds(0, _CL)],
            out_hbm.at[pl.ds(chunk_base(c), _CL)], so[p])
    for h in out_handles:
        if h is not None:
            h.wait()


def kernel(flattened_logits, mask):
    mask_i32 = mask.astype(jnp.int32).reshape(_B * _L)
    out = _restore(flattened_logits, mask_i32)
    return out.reshape(_B, _L)
